# Initial kernel scaffold; baseline (speedup 1.0000x reference)
import jax
import jax.numpy as jnp
from jax.experimental import pallas as pl


def kernel(h, edge_num, edge_index, edge_weight, W_sum, b_sum, W_mean, b_mean, W_num, b_num, g1, be1, g2, be2, g3, be3, Wl, bl, Wr, br, We, att, b_gat, g_att, be_att, g_e, be_e, W_fc, b_fc):
    raise NotImplementedError("write your pallas kernel here")


# speedup vs baseline: 10.0133x; 10.0133x over previous
import jax
import jax.numpy as jnp
from jax import lax
from jax.experimental import pallas as pl
from jax.experimental.pallas import tpu as pltpu
from jax.experimental.pallas import tpu_sc as plsc

N = 10000
E = 160000
NP = 10016
RPT = NP // 16
CH = 32
T1 = 5056
E1P = 32 * T1
CH2 = 64
T2 = 10688
E2P = 16 * T2
T0 = T2 // 2
GB = 4
NB = NP // GB

_f32 = jnp.float32
_i32 = jnp.int32


def _ln(x, g, b):
    m = jnp.mean(x, axis=-1, keepdims=True)
    v = jnp.mean((x - m) * (x - m), axis=-1, keepdims=True)
    return (x - m) / jnp.sqrt(v + 1e-5) * g + b


def _tc_a_body(h_ref, wcat_ref, ew2d_ref, xs_ref, ewo_ref):
    xs_ref[...] = jnp.dot(h_ref[...], wcat_ref[...],
                          preferred_element_type=_f32)
    ew = ew2d_ref[...]
    r = lax.broadcasted_iota(_i32, ew.shape, 0)
    c = lax.broadcasted_iota(_i32, ew.shape, 1)
    fl = r * ew.shape[1] + c
    real = fl < E
    mean = jnp.sum(jnp.where(real, ew, 0.0)) * (1.0 / E)
    ewo_ref[...] = jnp.where(real, ew, jnp.where(fl < E + N, mean, 0.0))


def _tc_a(hp, wcat, ew2d):
    return pl.pallas_call(
        _tc_a_body,
        out_shape=[jax.ShapeDtypeStruct((NP, 192), _f32),
                   jax.ShapeDtypeStruct(ew2d.shape, _f32)],
    )(hp, wcat, ew2d)


def _sc_cnt_body(dst_hbm, cntp_out, acc_sh, sem):
    def inner(dstv, ones_rows, zrow):
        c = lax.axis_index("c")
        s = lax.axis_index("s")
        w = s * 2 + c
        z16 = jnp.zeros((16,), _f32)
        one_first = jnp.where(lax.iota(_i32, 16) == 0, 1.0, 0.0)

        def zinit(i, _):
            ones_rows[i, :] = one_first
            zrow[i, :] = z16
            return 0
        lax.fori_loop(0, CH, zinit, 0)

        base0 = s * RPT
        def zacc(i, _):
            pltpu.sync_copy(zrow, acc_sh.at[pl.ds(base0 + i * CH, CH)])
            return 0
        lax.fori_loop(0, RPT // CH, zacc, 0)
        rem = RPT % CH
        if rem:
            pltpu.sync_copy(zrow.at[pl.ds(0, rem)],
                            acc_sh.at[pl.ds(base0 + RPT - rem, rem)])
        plsc.subcore_barrier()

        def chunk(k, _):
            base = w * T0 + k * CH
            pltpu.sync_copy(dst_hbm.at[pl.ds(base, CH)], dstv)
            pltpu.sync_copy(ones_rows, acc_sh.at[dstv], add=True)
            return 0
        lax.fori_loop(0, T0 // CH, chunk, 0)
        plsc.subcore_barrier()

        pltpu.sync_copy(acc_sh.at[pl.ds(base0, RPT)],
                        cntp_out.at[c, pl.ds(base0, RPT)])

    pl.run_scoped(inner,
                  pltpu.VMEM((CH,), _i32), pltpu.VMEM((CH, 16), _f32),
                  pltpu.VMEM((CH, 16), _f32))


def _sc_cnt(dstp):
    mesh = plsc.VectorSubcoreMesh(core_axis_name="c", subcore_axis_name="s")
    return pl.kernel(
        _sc_cnt_body,
        out_type=jax.ShapeDtypeStruct((2, NP, 16), _f32),
        mesh=mesh,
        scratch_types=[
            pltpu.VMEM_SHARED((NP, 16), _f32),
            pltpu.SemaphoreType.DMA,
        ],
        compiler_params=pltpu.CompilerParams(
            use_tc_tiling_on_sc=False, needs_layout_passes=False),
    )(dstp)


def _sc_gcn_body(xs_hbm, src_hbm, dst_hbm, ew_hbm, accp_out, acc_sh, sem):
    def inner(srcv, dstv, ewv, rows):
        c = lax.axis_index("c")
        s = lax.axis_index("s")
        w = s * 2 + c
        z16 = jnp.zeros((16,), _f32)

        def zrows(i, _):
            for j in range(12):
                rows[i, pl.ds(j * 16, 16)] = z16
            return 0
        lax.fori_loop(0, CH, zrows, 0)

        base0 = s * RPT
        def zacc(i, _):
            pltpu.sync_copy(rows, acc_sh.at[pl.ds(base0 + i * CH, CH)])
            return 0
        lax.fori_loop(0, RPT // CH, zacc, 0)
        rem = RPT % CH
        if rem:
            pltpu.sync_copy(rows.at[pl.ds(0, rem)],
                            acc_sh.at[pl.ds(base0 + RPT - rem, rem)])
        plsc.subcore_barrier()

        def chunk(k, _):
            base = w * T1 + k * CH
            pltpu.sync_copy(src_hbm.at[pl.ds(base, CH)], srcv)
            pltpu.sync_copy(dst_hbm.at[pl.ds(base, CH)], dstv)
            pltpu.sync_copy(ew_hbm.at[pl.ds(base, CH)], ewv)
            pltpu.async_copy(xs_hbm.at[srcv], rows, sem).wait()

            def edge(e, _2):
                ewe = plsc.load_gather(ewv, [jnp.full((16,), e, _i32)])
                for j in range(8):
                    sl = pl.ds(j * 16, 16)
                    rows[e, sl] = rows[e, sl] * ewe
                return 0
            lax.fori_loop(0, CH, edge, 0)
            pltpu.sync_copy(rows, acc_sh.at[dstv], add=True)
            return 0
        lax.fori_loop(0, T1 // CH, chunk, 0)
        plsc.subcore_barrier()

        pltpu.sync_copy(acc_sh.at[pl.ds(base0, RPT)],
                        accp_out.at[c, pl.ds(base0, RPT)])

    pl.run_scoped(inner,
                  pltpu.VMEM((CH,), _i32), pltpu.VMEM((CH,), _i32),
                  pltpu.VMEM((CH,), _f32), pltpu.VMEM((CH, 192), _f32))


def _sc_gcn(xs, srcp, dstp, ewp):
    mesh = plsc.VectorSubcoreMesh(core_axis_name="c", subcore_axis_name="s")
    return pl.kernel(
        _sc_gcn_body,
        out_type=jax.ShapeDtypeStruct((2, NP, 192), _f32),
        mesh=mesh,
        scratch_types=[
            pltpu.VMEM_SHARED((NP, 192), _f32),
            pltpu.SemaphoreType.DMA,
        ],
        compiler_params=pltpu.CompilerParams(
            use_tc_tiling_on_sc=False, needs_layout_passes=False),
    )(xs, srcp, dstp, ewp)


def _sc_gata_body(xl_hbm, xr_hbm, srcg_hbm, dstg_hbm, ew_hbm, att_hbm,
                  we_hbm, denp_out, ex_out, acc_sh, sem, sem2):
    def inner(srcv, dstv, ewv, xlr, xrr, attv, wev, exrows):
        c = lax.axis_index("c")
        s = lax.axis_index("s")
        z16 = jnp.zeros((16,), _f32)
        m0 = lax.iota(_i32, 16) == 0

        pltpu.sync_copy(att_hbm.at[c], attv)
        pltpu.sync_copy(we_hbm.at[c], wev)

        def zex(i, _):
            exrows[i, :] = z16
            return 0
        lax.fori_loop(0, CH2, zex, 0)

        base0 = s * RPT
        def zacc(i, _):
            pltpu.sync_copy(exrows, acc_sh.at[pl.ds(base0 + i * CH2, CH2)])
            return 0
        lax.fori_loop(0, RPT // CH2, zacc, 0)
        rem = RPT % CH2
        if rem:
            pltpu.sync_copy(exrows.at[pl.ds(0, rem)],
                            acc_sh.at[pl.ds(base0 + RPT - rem, rem)])
        plsc.subcore_barrier()

        def chunk(k, _):
            base = s * T2 + k * CH2
            pltpu.sync_copy(srcg_hbm.at[c, pl.ds(base, CH2)], srcv)
            pltpu.sync_copy(dstg_hbm.at[c, pl.ds(base, CH2)], dstv)
            pltpu.sync_copy(ew_hbm.at[pl.ds(base, CH2)], ewv)
            a1 = pltpu.async_copy(xl_hbm.at[srcv], xlr, sem)
            a2 = pltpu.async_copy(xr_hbm.at[dstv], xrr, sem2)
            a1.wait()
            a2.wait()

            def edge(e, _2):
                idx16 = jnp.full((16,), e, _i32)
                ewe = plsc.load_gather(ewv, [idx16])
                acc16 = z16
                for j in range(12):
                    sl = pl.ds(j * 16, 16)
                    r = xlr[e, sl] + xrr[e, sl] + ewe * wev[sl]
                    sj = jnp.maximum(r, r * 0.2)
                    acc16 = acc16 + sj * attv[sl]
                exv = jnp.exp(jnp.full((16,), jnp.sum(acc16)))
                exrows[e, :] = jnp.where(m0, exv, 0.0)
                plsc.store_scatter(ewv, [idx16], exv, mask=m0)
                return 0
            lax.fori_loop(0, CH2, edge, 0)
            pltpu.sync_copy(ewv, ex_out.at[c, pl.ds(base, CH2)])
            for g in range(CH2 // 16):
                sl = pl.ds(g * 16, 16)
                dstv[sl] = lax.shift_right_logical(dstv[sl], 1)
            pltpu.sync_copy(exrows, acc_sh.at[dstv], add=True)
            return 0
        lax.fori_loop(0, T2 // CH2, chunk, 0)
        plsc.subcore_barrier()

        pltpu.sync_copy(acc_sh.at[pl.ds(base0, RPT)],
                        denp_out.at[c, pl.ds(base0, RPT)])

    pl.run_scoped(inner,
                  pltpu.VMEM((CH2,), _i32), pltpu.VMEM((CH2,), _i32),
                  pltpu.VMEM((CH2,), _f32),
                  pltpu.VMEM((CH2, 192), _f32), pltpu.VMEM((CH2, 192), _f32),
                  pltpu.VMEM((192,), _f32), pltpu.VMEM((192,), _f32),
                  pltpu.VMEM((CH2, 16), _f32))


def _sc_gata(xl2t, xr2t, srcg, dstg, ewflat, att2, we2):
    mesh = plsc.VectorSubcoreMesh(core_axis_name="c", subcore_axis_name="s")
    return pl.kernel(
        _sc_gata_body,
        out_type=(jax.ShapeDtypeStruct((2, NP, 16), _f32),
                  jax.ShapeDtypeStruct((2, E2P), _f32)),
        mesh=mesh,
        scratch_types=[
            pltpu.VMEM_SHARED((NP, 16), _f32),
            pltpu.SemaphoreType.DMA,
            pltpu.SemaphoreType.DMA,
        ],
        compiler_params=pltpu.CompilerParams(
            use_tc_tiling_on_sc=False, needs_layout_passes=False),
    )(xl2t, xr2t, srcg, dstg, ewflat, att2, we2)


def _sc_gatb_body(xl_hbm, srcg_hbm, dstg_hbm, ex_hbm, msgp_out, acc_sh, sem):
    def inner(srcv, dstv, exv, rows):
        c = lax.axis_index("c")
        s = lax.axis_index("s")
        z16 = jnp.zeros((16,), _f32)

        def zrows(i, _):
            for j in range(12):
                rows[i, pl.ds(j * 16, 16)] = z16
            return 0
        lax.fori_loop(0, CH, zrows, 0)

        base0 = s * RPT
        def zacc(i, _):
            pltpu.sync_copy(rows, acc_sh.at[pl.ds(base0 + i * CH, CH)])
            return 0
        lax.fori_loop(0, RPT // CH, zacc, 0)
        rem = RPT % CH
        if rem:
            pltpu.sync_copy(rows.at[pl.ds(0, rem)],
                            acc_sh.at[pl.ds(base0 + RPT - rem, rem)])
        plsc.subcore_barrier()

        def chunk(k, _):
            base = s * T2 + k * CH
            pltpu.sync_copy(srcg_hbm.at[c, pl.ds(base, CH)], srcv)
            pltpu.sync_copy(dstg_hbm.at[c, pl.ds(base, CH)], dstv)
            pltpu.sync_copy(ex_hbm.at[c, pl.ds(base, CH)], exv)
            pltpu.async_copy(xl_hbm.at[srcv], rows, sem).wait()

            def edge(e, _2):
                exe = plsc.load_gather(exv, [jnp.full((16,), e, _i32)])
                for j in range(12):
                    sl = pl.ds(j * 16, 16)
                    rows[e, sl] = rows[e, sl] * exe
                return 0
            lax.fori_loop(0, CH, edge, 0)
            for g in range(CH // 16):
                sl = pl.ds(g * 16, 16)
                dstv[sl] = lax.shift_right_logical(dstv[sl], 1)
            pltpu.sync_copy(rows, acc_sh.at[dstv], add=True)
            return 0
        lax.fori_loop(0, T2 // CH, chunk, 0)
        plsc.subcore_barrier()

        pltpu.sync_copy(acc_sh.at[pl.ds(base0, RPT)],
                        msgp_out.at[c, pl.ds(base0, RPT)])

    pl.run_scoped(inner,
                  pltpu.VMEM((CH,), _i32), pltpu.VMEM((CH,), _i32),
                  pltpu.VMEM((CH,), _f32), pltpu.VMEM((CH, 192), _f32))


def _sc_gatb(xl2t, srcg, dstg, exbuf):
    mesh = plsc.VectorSubcoreMesh(core_axis_name="c", subcore_axis_name="s")
    return pl.kernel(
        _sc_gatb_body,
        out_type=jax.ShapeDtypeStruct((2, NP, 192), _f32),
        mesh=mesh,
        scratch_types=[
            pltpu.VMEM_SHARED((NP, 192), _f32),
            pltpu.SemaphoreType.DMA,
        ],
        compiler_params=pltpu.CompilerParams(
            use_tc_tiling_on_sc=False, needs_layout_passes=False),
    )(xl2t, srcg, dstg, exbuf)


def _tc_b_body(accp, cntp, bsum, g1r, be1r, bmean, g2r, be2r, bnum, g3r,
               be3r, wl, blr, wr, brr, xlt, xrt, cnts):
    acc = accp[0] + accp[1]
    cnt2 = cntp[0, :, 0] + cntp[1, :, 0]
    cnt = jnp.maximum(cnt2 - 1.0, 1.0)
    h1 = _ln(acc[:, :64] + bsum[0], g1r[0], be1r[0])
    h2 = _ln(acc[:, 64:128] / cnt[:, None] + bmean[0], g2r[0], be2r[0])
    h3 = _ln(acc[:, 128:192] + bnum[0], g3r[0], be3r[0])
    x = jnp.concatenate([h1, h2, h3], axis=-1)
    xlt[...] = jnp.dot(x, wl[...], preferred_element_type=_f32) + blr[0]
    xrt[...] = jnp.dot(x, wr[...], preferred_element_type=_f32) + brr[0]
    cnts[...] = jnp.broadcast_to(cnt2[:, None], cnts.shape)


def _tc_b(accp, cntp, consts):
    full = lambda shape: pl.BlockSpec(shape, lambda i: (0,) * len(shape))
    return pl.pallas_call(
        _tc_b_body,
        grid=(GB,),
        in_specs=[
            pl.BlockSpec((2, NB, 192), lambda i: (0, i, 0)),
            pl.BlockSpec((2, NB, 16), lambda i: (0, i, 0)),
            full((1, 64)), full((1, 64)), full((1, 64)), full((1, 64)),
            full((1, 64)), full((1, 64)), full((1, 64)), full((1, 64)),
            full((1, 64)),
            full((192, 384)), full((1, 384)),
            full((192, 384)), full((1, 384)),
        ],
        out_specs=[
            pl.BlockSpec((NB, 384), lambda i: (i, 0)),
            pl.BlockSpec((NB, 384), lambda i: (i, 0)),
            pl.BlockSpec((NB, 8), lambda i: (i, 0)),
        ],
        out_shape=[jax.ShapeDtypeStruct((NP, 384), _f32),
                   jax.ShapeDtypeStruct((NP, 384), _f32),
                   jax.ShapeDtypeStruct((NP, 8), _f32)],
    )(accp, cntp, *consts)


def _tc_c_body(msgp, denp, cnts, enum_r, bgat, gatt, beatt, ger, beer,
               wfc1, wfc2, bfc, y):
    den0 = denp[0, :, 0][:, None] + 1e-16
    den1 = denp[1, :, 0][:, None] + 1e-16
    cnt2 = jnp.maximum(cnts[:, 0][:, None], 1.0)
    agg = (msgp[0] / den0 + msgp[1] / den1) * (0.5 / cnt2)
    out = _ln(agg + bgat[0], gatt[0], beatt[0])
    en = _ln(enum_r[...], ger[0], beer[0])
    y[...] = (jnp.dot(out, wfc1[...], preferred_element_type=_f32)
              + jnp.dot(en, wfc2[...], preferred_element_type=_f32)
              + bfc[0])


def _tc_c(msgp, denp, cnts, enump, consts):
    full = lambda shape: pl.BlockSpec(shape, lambda i: (0,) * len(shape))
    return pl.pallas_call(
        _tc_c_body,
        grid=(GB,),
        in_specs=[
            pl.BlockSpec((2, NB, 192), lambda i: (0, i, 0)),
            pl.BlockSpec((2, NB, 16), lambda i: (0, i, 0)),
            pl.BlockSpec((NB, 8), lambda i: (i, 0)),
            pl.BlockSpec((NB, 5), lambda i: (i, 0)),
            full((1, 192)), full((1, 192)), full((1, 192)),
            full((1, 5)), full((1, 5)),
            full((192, 5)), full((5, 5)), full((1, 5)),
        ],
        out_specs=pl.BlockSpec((NB, 5), lambda i: (i, 0)),
        out_shape=jax.ShapeDtypeStruct((NP, 5), _f32),
    )(msgp, denp, cnts, enump, *consts)


def kernel(h, edge_num, edge_index, edge_weight, W_sum, b_sum, W_mean,
           b_mean, W_num, b_num, g1, be1, g2, be2, g3, be3, Wl, bl, Wr, br,
           We, att, b_gat, g_att, be_att, g_e, be_e, W_fc, b_fc):
    src = edge_index[0]
    dst = edge_index[1]
    r1 = lambda a: a.reshape(1, -1)

    hp = jnp.pad(h, ((0, NP - N), (0, 0)))
    wcat = jnp.concatenate([W_sum, W_mean, W_num], axis=1)
    ew2d = jnp.pad(edge_weight, (0, E2P - E)).reshape(E2P // 128, 128)
    xs, ewo = _tc_a(hp, wcat, ew2d)

    loop = jnp.arange(N, dtype=src.dtype)
    src2 = jnp.concatenate([src, loop])
    dst2 = jnp.concatenate([dst, loop])
    pad2 = E2P - (E + N)
    srcp2 = jnp.pad(src2, (0, pad2))
    dst2p = jnp.pad(dst2, (0, pad2), constant_values=N)
    srcg = jnp.stack([srcp2 * 2, srcp2 * 2 + 1])
    dstg = jnp.stack([dst2p * 2, dst2p * 2 + 1])

    cntp = _sc_cnt(dst2p)

    srcp1 = jnp.pad(src, (0, E1P - E))
    dstp1 = jnp.pad(dst, (0, E1P - E), constant_values=N)
    ewp1 = jnp.pad(edge_weight, (0, E1P - E))
    accp = _sc_gcn(xs, srcp1, dstp1, ewp1)

    xlt, xrt, cnts = _tc_b(
        accp, cntp,
        (r1(b_sum), r1(g1), r1(be1), r1(b_mean), r1(g2), r1(be2),
         r1(b_num), r1(g3), r1(be3), Wl, r1(bl), Wr, r1(br)))

    xl2t = xlt.reshape(2 * NP, 192)
    xr2t = xrt.reshape(2 * NP, 192)
    denp, exbuf = _sc_gata(xl2t, xr2t, srcg, dstg, ewo.reshape(E2P), att,
                           We.reshape(2, 192))
    msgp = _sc_gatb(xl2t, srcg, dstg, exbuf)

    y = _tc_c(msgp, denp, cnts, jnp.pad(edge_num, ((0, NP - N), (0, 0))),
              (r1(b_gat), r1(g_att), r1(be_att), r1(g_e), r1(be_e),
               W_fc[:192], W_fc[192:], r1(b_fc)))
    return y[:N]


# speedup vs baseline: 14.2778x; 1.4259x over previous
import jax
import jax.numpy as jnp
from jax import lax
from jax.experimental import pallas as pl
from jax.experimental.pallas import tpu as pltpu
from jax.experimental.pallas import tpu_sc as plsc

N = 10000
E = 160000
NP = 10016
RPT = NP // 16

CH0 = 32
CH1 = 16
G1 = 32
T1 = 5120
E1P = 32 * T1

CHA = 32
GA = 24
CHB = 16
GB2 = 48
T2 = 10752
E2P = 16 * T2
T0 = E2P // 32

GB = 4
NB = NP // GB

_f32 = jnp.float32
_i32 = jnp.int32


def _ln(x, g, b):
    m = jnp.mean(x, axis=-1, keepdims=True)
    v = jnp.mean((x - m) * (x - m), axis=-1, keepdims=True)
    return (x - m) / jnp.sqrt(v + 1e-5) * g + b


def _tc_a_body(h_ref, wcat_ref, ew2d_ref, xs_ref, ewo_ref):
    xs_ref[...] = jnp.dot(h_ref[...], wcat_ref[...],
                          preferred_element_type=_f32)
    ew = ew2d_ref[...]
    r = lax.broadcasted_iota(_i32, ew.shape, 0)
    c = lax.broadcasted_iota(_i32, ew.shape, 1)
    fl = r * ew.shape[1] + c
    real = fl < E
    mean = jnp.sum(jnp.where(real, ew, 0.0)) * (1.0 / E)
    ewo_ref[...] = jnp.where(real, ew, jnp.where(fl < E + N, mean, 0.0))


def _tc_a(hp, wcat, ew2d):
    return pl.pallas_call(
        _tc_a_body,
        out_shape=[jax.ShapeDtypeStruct((NP, 192), _f32),
                   jax.ShapeDtypeStruct(ew2d.shape, _f32)],
    )(hp, wcat, ew2d)


def _sc_cnt_body(dst_hbm, cntp_out, acc_sh, sem):
    def inner(dstv, ones_rows, zrow):
        c = lax.axis_index("c")
        s = lax.axis_index("s")
        w = s * 2 + c
        z16 = jnp.zeros((16,), _f32)
        one_first = jnp.where(lax.iota(_i32, 16) == 0, 1.0, 0.0)

        def zinit(i, _):
            ones_rows[i, :] = one_first
            zrow[i, :] = z16
            return 0
        lax.fori_loop(0, CH0, zinit, 0)

        base0 = s * RPT
        def zacc(i, _):
            pltpu.sync_copy(zrow, acc_sh.at[pl.ds(base0 + i * CH0, CH0)])
            return 0
        lax.fori_loop(0, RPT // CH0, zacc, 0)
        rem = RPT % CH0
        if rem:
            pltpu.sync_copy(zrow.at[pl.ds(0, rem)],
                            acc_sh.at[pl.ds(base0 + RPT - rem, rem)])
        plsc.subcore_barrier()

        def chunk(k, _):
            base = w * T0 + k * CH0
            pltpu.sync_copy(dst_hbm.at[pl.ds(base, CH0)], dstv)
            pltpu.sync_copy(ones_rows, acc_sh.at[dstv], add=True)
            return 0
        lax.fori_loop(0, T0 // CH0, chunk, 0)
        plsc.subcore_barrier()

        pltpu.sync_copy(acc_sh.at[pl.ds(base0, RPT)],
                        cntp_out.at[c, pl.ds(base0, RPT)])

    pl.run_scoped(inner,
                  pltpu.VMEM((CH0,), _i32), pltpu.VMEM((CH0, 16), _f32),
                  pltpu.VMEM((CH0, 16), _f32))


def _sc_cnt(dstp):
    mesh = plsc.VectorSubcoreMesh(core_axis_name="c", subcore_axis_name="s")
    return pl.kernel(
        _sc_cnt_body,
        out_type=jax.ShapeDtypeStruct((2, NP, 16), _f32),
        mesh=mesh,
        scratch_types=[
            pltpu.VMEM_SHARED((NP, 16), _f32),
            pltpu.SemaphoreType.DMA,
        ],
        compiler_params=pltpu.CompilerParams(
            use_tc_tiling_on_sc=False, needs_layout_passes=False),
    )(dstp)


def _sc_gcn_body(xs_hbm, src_hbm, dst_hbm, ew_hbm, accp_out, acc_sh,
                 sg0, sg1, ss0, ss1):
    def inner(prefs, prefd, prefe, rows0, rows1, dsts0, dsts1):
        c = lax.axis_index("c")
        s = lax.axis_index("s")
        w = s * 2 + c
        z16 = jnp.zeros((16,), _f32)
        rows = (rows0, rows1)
        dsts = (dsts0, dsts1)
        semg = (sg0, sg1)
        sems = (ss0, ss1)
        nch = T1 // CH1
        tbase = w * T1

        def zrows(i, _):
            for j in range(12):
                rows0[i, pl.ds(j * 16, 16)] = z16
            return 0
        lax.fori_loop(0, CH1, zrows, 0)
        base0 = s * RPT
        def zacc(i, _):
            pltpu.sync_copy(rows0, acc_sh.at[pl.ds(base0 + i * CH1, CH1)])
            return 0
        lax.fori_loop(0, RPT // CH1, zacc, 0)
        rem = RPT % CH1
        if rem:
            pltpu.sync_copy(rows0.at[pl.ds(0, rem)],
                            acc_sh.at[pl.ds(base0 + RPT - rem, rem)])
        plsc.subcore_barrier()

        def refill(nxt):
            gb = tbase + nxt * CH1
            hb = lax.rem(nxt, 2 * G1) * CH1
            pltpu.sync_copy(src_hbm.at[pl.ds(gb, G1 * CH1)],
                            prefs.at[pl.ds(hb, G1 * CH1)])
            pltpu.sync_copy(dst_hbm.at[pl.ds(gb, G1 * CH1)],
                            prefd.at[pl.ds(hb, G1 * CH1)])
            pltpu.sync_copy(ew_hbm.at[pl.ds(gb, G1 * CH1)],
                            prefe.at[pl.ds(hb, G1 * CH1)])

        def issue_gather(nxt, b):
            off = lax.rem(nxt, 2 * G1) * CH1
            pltpu.async_copy(xs_hbm.at[prefs.at[pl.ds(off, CH1)]],
                             rows[b], semg[b])

        refill(0)
        issue_gather(0, 0)

        def super_chunk(m, _):
            for b in range(2):
                k = 2 * m + b
                b2 = 1 - b
                pltpu.make_async_copy(
                    xs_hbm.at[prefs.at[pl.ds(0, CH1)]], rows[b],
                    semg[b]).wait()
                nxt = k + 1

                @pl.when(nxt < nch)
                def _issue():
                    @pl.when(lax.rem(nxt, G1) == 0)
                    def _refill():
                        refill(nxt)
                    def _drain():
                        pltpu.make_async_copy(
                            rows[b2], acc_sh.at[dsts[b2]], sems[b2]).wait()
                    if b == 1:
                        _drain()
                    else:
                        pl.when(m > 0)(_drain)
                    issue_gather(nxt, b2)

                off = lax.rem(k, 2 * G1) * CH1

                def edge(e, _2):
                    ewe = plsc.load_gather(
                        prefe, [jnp.full((16,), off + e, _i32)])
                    for j in range(8):
                        sl = pl.ds(j * 16, 16)
                        rows[b][e, sl] = rows[b][e, sl] * ewe
                    return 0
                lax.fori_loop(0, CH1, edge, 0)
                dsts[b][pl.ds(0, 16)] = prefd[pl.ds(off, 16)]
                pltpu.async_copy(rows[b], acc_sh.at[dsts[b]], sems[b],
                                 add=True)
            return 0
        lax.fori_loop(0, nch // 2, super_chunk, 0)
        for b in range(2):
            pltpu.make_async_copy(rows[b], acc_sh.at[dsts[b]],
                                  sems[b]).wait()
        plsc.subcore_barrier()

        pltpu.sync_copy(acc_sh.at[pl.ds(base0, RPT)],
                        accp_out.at[c, pl.ds(base0, RPT)])

    pl.run_scoped(inner,
                  pltpu.VMEM((2 * G1 * CH1,), _i32),
                  pltpu.VMEM((2 * G1 * CH1,), _i32),
                  pltpu.VMEM((2 * G1 * CH1,), _f32),
                  pltpu.VMEM((CH1, 192), _f32), pltpu.VMEM((CH1, 192), _f32),
                  pltpu.VMEM((16,), _i32), pltpu.VMEM((16,), _i32))


def _sc_gcn(xs, srcp, dstp, ewp):
    mesh = plsc.VectorSubcoreMesh(core_axis_name="c", subcore_axis_name="s")
    return pl.kernel(
        _sc_gcn_body,
        out_type=jax.ShapeDtypeStruct((2, NP, 192), _f32),
        mesh=mesh,
        scratch_types=[
            pltpu.VMEM_SHARED((NP, 192), _f32),
            pltpu.SemaphoreType.DMA, pltpu.SemaphoreType.DMA,
            pltpu.SemaphoreType.DMA, pltpu.SemaphoreType.DMA,
        ],
        compiler_params=pltpu.CompilerParams(
            use_tc_tiling_on_sc=False, needs_layout_passes=False),
    )(xs, srcp, dstp, ewp)


def _sc_gata_body(xl_hbm, xr_hbm, srcg_hbm, dstg_hbm, ew_hbm, att_hbm,
                  we_hbm, denp_out, ex_out, acc_sh,
                  sgl0, sgl1, sgr0, sgr1, ss0, ss1):
    def inner(prefs, prefd, prefe, exall, xlr0, xlr1, xrr0, xrr1,
              exr0, exr1, dsts0, dsts1, attv, wev):
        c = lax.axis_index("c")
        s = lax.axis_index("s")
        z16 = jnp.zeros((16,), _f32)
        m0 = lax.iota(_i32, 16) == 0
        xlr = (xlr0, xlr1)
        xrr = (xrr0, xrr1)
        exr = (exr0, exr1)
        dsts = (dsts0, dsts1)
        semgl = (sgl0, sgl1)
        semgr = (sgr0, sgr1)
        sems = (ss0, ss1)
        nch = T2 // CHA
        tbase = s * T2

        pltpu.sync_copy(att_hbm.at[c], attv)
        pltpu.sync_copy(we_hbm.at[c], wev)

        def zex(i, _):
            exr0[i, :] = z16
            return 0
        lax.fori_loop(0, CHA, zex, 0)
        base0 = s * RPT
        def zacc(i, _):
            pltpu.sync_copy(exr0, acc_sh.at[pl.ds(base0 + i * CHA, CHA)])
            return 0
        lax.fori_loop(0, RPT // CHA, zacc, 0)
        rem = RPT % CHA
        if rem:
            pltpu.sync_copy(exr0.at[pl.ds(0, rem)],
                            acc_sh.at[pl.ds(base0 + RPT - rem, rem)])
        plsc.subcore_barrier()

        def refill(nxt):
            gb = tbase + nxt * CHA
            hb = lax.rem(nxt, 2 * GA) * CHA
            pltpu.sync_copy(srcg_hbm.at[c, pl.ds(gb, GA * CHA)],
                            prefs.at[pl.ds(hb, GA * CHA)])
            pltpu.sync_copy(dstg_hbm.at[c, pl.ds(gb, GA * CHA)],
                            prefd.at[pl.ds(hb, GA * CHA)])
            pltpu.sync_copy(ew_hbm.at[pl.ds(gb, GA * CHA)],
                            prefe.at[pl.ds(hb, GA * CHA)])

        def issue_gather(nxt, b):
            off = lax.rem(nxt, 2 * GA) * CHA
            pltpu.async_copy(xl_hbm.at[prefs.at[pl.ds(off, CHA)]],
                             xlr[b], semgl[b])
            pltpu.async_copy(xr_hbm.at[prefd.at[pl.ds(off, CHA)]],
                             xrr[b], semgr[b])

        refill(0)
        issue_gather(0, 0)

        def super_chunk(m, _):
            for b in range(2):
                k = 2 * m + b
                b2 = 1 - b
                pltpu.make_async_copy(
                    xl_hbm.at[prefs.at[pl.ds(0, CHA)]], xlr[b],
                    semgl[b]).wait()
                pltpu.make_async_copy(
                    xr_hbm.at[prefd.at[pl.ds(0, CHA)]], xrr[b],
                    semgr[b]).wait()
                nxt = k + 1

                @pl.when(nxt < nch)
                def _issue():
                    @pl.when(lax.rem(nxt, GA) == 0)
                    def _refill():
                        refill(nxt)
                    def _drain():
                        pltpu.make_async_copy(
                            exr[b2], acc_sh.at[dsts[b2]], sems[b2]).wait()
                    if b == 1:
                        _drain()
                    else:
                        pl.when(m > 0)(_drain)
                    issue_gather(nxt, b2)

                off = lax.rem(k, 2 * GA) * CHA
                offx = lax.rem(k, GA) * CHA

                def edge(e, _2):
                    idx16 = jnp.full((16,), off + e, _i32)
                    ewe = plsc.load_gather(prefe, [idx16])
                    acc16 = z16
                    for j in range(12):
                        sl = pl.ds(j * 16, 16)
                        r = xlr[b][e, sl] + xrr[b][e, sl] + ewe * wev[sl]
                        sj = jnp.maximum(r, r * 0.2)
                        acc16 = acc16 + sj * attv[sl]
                    exv = jnp.exp(jnp.full((16,), jnp.sum(acc16)))
                    exr[b][e, :] = jnp.where(m0, exv, 0.0)
                    plsc.store_scatter(
                        exall, [jnp.full((16,), offx + e, _i32)], exv,
                        mask=m0)
                    return 0
                lax.fori_loop(0, CHA, edge, 0)
                for g in range(CHA // 16):
                    sl = pl.ds(g * 16, 16)
                    dsts[b][sl] = lax.shift_right_logical(
                        prefd[pl.ds(off + g * 16, 16)], 1)
                pltpu.async_copy(exr[b], acc_sh.at[dsts[b]], sems[b],
                                 add=True)

                @pl.when(lax.rem(k + 1, GA) == 0)
                def _flush():
                    gb = tbase + (k + 1 - GA) * CHA
                    pltpu.sync_copy(exall,
                                    ex_out.at[c, pl.ds(gb, GA * CHA)])
            return 0
        lax.fori_loop(0, nch // 2, super_chunk, 0)
        for b in range(2):
            pltpu.make_async_copy(exr[b], acc_sh.at[dsts[b]],
                                  sems[b]).wait()
        plsc.subcore_barrier()

        pltpu.sync_copy(acc_sh.at[pl.ds(base0, RPT)],
                        denp_out.at[c, pl.ds(base0, RPT)])

    pl.run_scoped(inner,
                  pltpu.VMEM((2 * GA * CHA,), _i32),
                  pltpu.VMEM((2 * GA * CHA,), _i32),
                  pltpu.VMEM((2 * GA * CHA,), _f32),
                  pltpu.VMEM((GA * CHA,), _f32),
                  pltpu.VMEM((CHA, 192), _f32), pltpu.VMEM((CHA, 192), _f32),
                  pltpu.VMEM((CHA, 192), _f32), pltpu.VMEM((CHA, 192), _f32),
                  pltpu.VMEM((CHA, 16), _f32), pltpu.VMEM((CHA, 16), _f32),
                  pltpu.VMEM((CHA,), _i32), pltpu.VMEM((CHA,), _i32),
                  pltpu.VMEM((192,), _f32), pltpu.VMEM((192,), _f32))


def _sc_gata(xl2t, xr2t, srcg, dstg, ewflat, att2, we2):
    mesh = plsc.VectorSubcoreMesh(core_axis_name="c", subcore_axis_name="s")
    return pl.kernel(
        _sc_gata_body,
        out_type=(jax.ShapeDtypeStruct((2, NP, 16), _f32),
                  jax.ShapeDtypeStruct((2, E2P), _f32)),
        mesh=mesh,
        scratch_types=[
            pltpu.VMEM_SHARED((NP, 16), _f32),
            pltpu.SemaphoreType.DMA, pltpu.SemaphoreType.DMA,
            pltpu.SemaphoreType.DMA, pltpu.SemaphoreType.DMA,
            pltpu.SemaphoreType.DMA, pltpu.SemaphoreType.DMA,
        ],
        compiler_params=pltpu.CompilerParams(
            use_tc_tiling_on_sc=False, needs_layout_passes=False),
    )(xl2t, xr2t, srcg, dstg, ewflat, att2, we2)


def _sc_gatb_body(xl_hbm, srcg_hbm, dstg_hbm, ex_hbm, msgp_out, acc_sh,
                  sg0, sg1, ss0, ss1):
    def inner(prefs, prefd, prefe, rows0, rows1, dsts0, dsts1):
        c = lax.axis_index("c")
        s = lax.axis_index("s")
        z16 = jnp.zeros((16,), _f32)
        rows = (rows0, rows1)
        dsts = (dsts0, dsts1)
        semg = (sg0, sg1)
        sems = (ss0, ss1)
        nch = T2 // CHB
        tbase = s * T2

        def zrows(i, _):
            for j in range(12):
                rows0[i, pl.ds(j * 16, 16)] = z16
            return 0
        lax.fori_loop(0, CHB, zrows, 0)
        base0 = s * RPT
        def zacc(i, _):
            pltpu.sync_copy(rows0, acc_sh.at[pl.ds(base0 + i * CHB, CHB)])
            return 0
        lax.fori_loop(0, RPT // CHB, zacc, 0)
        rem = RPT % CHB
        if rem:
            pltpu.sync_copy(rows0.at[pl.ds(0, rem)],
                            acc_sh.at[pl.ds(base0 + RPT - rem, rem)])
        plsc.subcore_barrier()

        def refill(nxt):
            gb = tbase + nxt * CHB
            hb = lax.rem(nxt, 2 * GB2) * CHB
            pltpu.sync_copy(srcg_hbm.at[c, pl.ds(gb, GB2 * CHB)],
                            prefs.at[pl.ds(hb, GB2 * CHB)])
            pltpu.sync_copy(dstg_hbm.at[c, pl.ds(gb, GB2 * CHB)],
                            prefd.at[pl.ds(hb, GB2 * CHB)])
            pltpu.sync_copy(ex_hbm.at[c, pl.ds(gb, GB2 * CHB)],
                            prefe.at[pl.ds(hb, GB2 * CHB)])

        def issue_gather(nxt, b):
            off = lax.rem(nxt, 2 * GB2) * CHB
            pltpu.async_copy(xl_hbm.at[prefs.at[pl.ds(off, CHB)]],
                             rows[b], semg[b])

        refill(0)
        issue_gather(0, 0)

        def super_chunk(m, _):
            for b in range(2):
                k = 2 * m + b
                b2 = 1 - b
                pltpu.make_async_copy(
                    xl_hbm.at[prefs.at[pl.ds(0, CHB)]], rows[b],
                    semg[b]).wait()
                nxt = k + 1

                @pl.when(nxt < nch)
                def _issue():
                    @pl.when(lax.rem(nxt, GB2) == 0)
                    def _refill():
                        refill(nxt)
                    def _drain():
                        pltpu.make_async_copy(
                            rows[b2], acc_sh.at[dsts[b2]], sems[b2]).wait()
                    if b == 1:
                        _drain()
                    else:
                        pl.when(m > 0)(_drain)
                    issue_gather(nxt, b2)

                off = lax.rem(k, 2 * GB2) * CHB

                def edge(e, _2):
                    exe = plsc.load_gather(
                        prefe, [jnp.full((16,), off + e, _i32)])
                    for j in range(12):
                        sl = pl.ds(j * 16, 16)
                        rows[b][e, sl] = rows[b][e, sl] * exe
                    return 0
                lax.fori_loop(0, CHB, edge, 0)
                dsts[b][pl.ds(0, 16)] = lax.shift_right_logical(
                    prefd[pl.ds(off, 16)], 1)
                pltpu.async_copy(rows[b], acc_sh.at[dsts[b]], sems[b],
                                 add=True)
            return 0
        lax.fori_loop(0, nch // 2, super_chunk, 0)
        for b in range(2):
            pltpu.make_async_copy(rows[b], acc_sh.at[dsts[b]],
                                  sems[b]).wait()
        plsc.subcore_barrier()

        pltpu.sync_copy(acc_sh.at[pl.ds(base0, RPT)],
                        msgp_out.at[c, pl.ds(base0, RPT)])

    pl.run_scoped(inner,
                  pltpu.VMEM((2 * GB2 * CHB,), _i32),
                  pltpu.VMEM((2 * GB2 * CHB,), _i32),
                  pltpu.VMEM((2 * GB2 * CHB,), _f32),
                  pltpu.VMEM((CHB, 192), _f32), pltpu.VMEM((CHB, 192), _f32),
                  pltpu.VMEM((16,), _i32), pltpu.VMEM((16,), _i32))


def _sc_gatb(xl2t, srcg, dstg, exbuf):
    mesh = plsc.VectorSubcoreMesh(core_axis_name="c", subcore_axis_name="s")
    return pl.kernel(
        _sc_gatb_body,
        out_type=jax.ShapeDtypeStruct((2, NP, 192), _f32),
        mesh=mesh,
        scratch_types=[
            pltpu.VMEM_SHARED((NP, 192), _f32),
            pltpu.SemaphoreType.DMA, pltpu.SemaphoreType.DMA,
            pltpu.SemaphoreType.DMA, pltpu.SemaphoreType.DMA,
        ],
        compiler_params=pltpu.CompilerParams(
            use_tc_tiling_on_sc=False, needs_layout_passes=False),
    )(xl2t, srcg, dstg, exbuf)


def _tc_b_body(accp, cntp, bsum, g1r, be1r, bmean, g2r, be2r, bnum, g3r,
               be3r, wl, blr, wr, brr, xlt, xrt, cnts):
    acc = accp[0] + accp[1]
    cnt2 = cntp[0, :, 0] + cntp[1, :, 0]
    cnt = jnp.maximum(cnt2 - 1.0, 1.0)
    h1 = _ln(acc[:, :64] + bsum[0], g1r[0], be1r[0])
    h2 = _ln(acc[:, 64:128] / cnt[:, None] + bmean[0], g2r[0], be2r[0])
    h3 = _ln(acc[:, 128:192] + bnum[0], g3r[0], be3r[0])
    x = jnp.concatenate([h1, h2, h3], axis=-1)
    xlt[...] = jnp.dot(x, wl[...], preferred_element_type=_f32) + blr[0]
    xrt[...] = jnp.dot(x, wr[...], preferred_element_type=_f32) + brr[0]
    cnts[...] = jnp.broadcast_to(cnt2[:, None], cnts.shape)


def _tc_b(accp, cntp, consts):
    full = lambda shape: pl.BlockSpec(shape, lambda i: (0,) * len(shape))
    return pl.pallas_call(
        _tc_b_body,
        grid=(GB,),
        in_specs=[
            pl.BlockSpec((2, NB, 192), lambda i: (0, i, 0)),
            pl.BlockSpec((2, NB, 16), lambda i: (0, i, 0)),
            full((1, 64)), full((1, 64)), full((1, 64)), full((1, 64)),
            full((1, 64)), full((1, 64)), full((1, 64)), full((1, 64)),
            full((1, 64)),
            full((192, 384)), full((1, 384)),
            full((192, 384)), full((1, 384)),
        ],
        out_specs=[
            pl.BlockSpec((NB, 384), lambda i: (i, 0)),
            pl.BlockSpec((NB, 384), lambda i: (i, 0)),
            pl.BlockSpec((NB, 8), lambda i: (i, 0)),
        ],
        out_shape=[jax.ShapeDtypeStruct((NP, 384), _f32),
                   jax.ShapeDtypeStruct((NP, 384), _f32),
                   jax.ShapeDtypeStruct((NP, 8), _f32)],
    )(accp, cntp, *consts)


def _tc_c_body(msgp, denp, cnts, enum_r, bgat, gatt, beatt, ger, beer,
               wfc1, wfc2, bfc, y):
    den0 = denp[0, :, 0][:, None] + 1e-16
    den1 = denp[1, :, 0][:, None] + 1e-16
    cnt2 = jnp.maximum(cnts[:, 0][:, None], 1.0)
    agg = (msgp[0] / den0 + msgp[1] / den1) * (0.5 / cnt2)
    out = _ln(agg + bgat[0], gatt[0], beatt[0])
    en = _ln(enum_r[...], ger[0], beer[0])
    y[...] = (jnp.dot(out, wfc1[...], preferred_element_type=_f32)
              + jnp.dot(en, wfc2[...], preferred_element_type=_f32)
              + bfc[0])


def _tc_c(msgp, denp, cnts, enump, consts):
    full = lambda shape: pl.BlockSpec(shape, lambda i: (0,) * len(shape))
    return pl.pallas_call(
        _tc_c_body,
        grid=(GB,),
        in_specs=[
            pl.BlockSpec((2, NB, 192), lambda i: (0, i, 0)),
            pl.BlockSpec((2, NB, 16), lambda i: (0, i, 0)),
            pl.BlockSpec((NB, 8), lambda i: (i, 0)),
            pl.BlockSpec((NB, 5), lambda i: (i, 0)),
            full((1, 192)), full((1, 192)), full((1, 192)),
            full((1, 5)), full((1, 5)),
            full((192, 5)), full((5, 5)), full((1, 5)),
        ],
        out_specs=pl.BlockSpec((NB, 5), lambda i: (i, 0)),
        out_shape=jax.ShapeDtypeStruct((NP, 5), _f32),
    )(msgp, denp, cnts, enump, *consts)


def kernel(h, edge_num, edge_index, edge_weight, W_sum, b_sum, W_mean,
           b_mean, W_num, b_num, g1, be1, g2, be2, g3, be3, Wl, bl, Wr, br,
           We, att, b_gat, g_att, be_att, g_e, be_e, W_fc, b_fc):
    src = edge_index[0]
    dst = edge_index[1]
    r1 = lambda a: a.reshape(1, -1)

    hp = jnp.pad(h, ((0, NP - N), (0, 0)))
    wcat = jnp.concatenate([W_sum, W_mean, W_num], axis=1)
    ew2d = jnp.pad(edge_weight, (0, E2P - E)).reshape(E2P // 128, 128)
    xs, ewo = _tc_a(hp, wcat, ew2d)

    loop = jnp.arange(N, dtype=src.dtype)
    src2 = jnp.concatenate([src, loop])
    dst2 = jnp.concatenate([dst, loop])
    pad2 = E2P - (E + N)
    srcp2 = jnp.pad(src2, (0, pad2))
    dst2p = jnp.pad(dst2, (0, pad2), constant_values=N)
    srcg = jnp.stack([srcp2 * 2, srcp2 * 2 + 1])
    dstg = jnp.stack([dst2p * 2, dst2p * 2 + 1])

    cntp = _sc_cnt(dst2p)

    srcp1 = jnp.pad(src, (0, E1P - E))
    dstp1 = jnp.pad(dst, (0, E1P - E), constant_values=N)
    ewp1 = jnp.pad(edge_weight, (0, E1P - E))
    accp = _sc_gcn(xs, srcp1, dstp1, ewp1)

    xlt, xrt, cnts = _tc_b(
        accp, cntp,
        (r1(b_sum), r1(g1), r1(be1), r1(b_mean), r1(g2), r1(be2),
         r1(b_num), r1(g3), r1(be3), Wl, r1(bl), Wr, r1(br)))

    xl2t = xlt.reshape(2 * NP, 192)
    xr2t = xrt.reshape(2 * NP, 192)
    denp, exbuf = _sc_gata(xl2t, xr2t, srcg, dstg, ewo.reshape(E2P), att,
                           We.reshape(2, 192))
    msgp = _sc_gatb(xl2t, srcg, dstg, exbuf)

    y = _tc_c(msgp, denp, cnts, jnp.pad(edge_num, ((0, NP - N), (0, 0))),
              (r1(b_gat), r1(g_att), r1(be_att), r1(g_e), r1(be_e),
               W_fc[:192], W_fc[192:], r1(b_fc)))
    return y[:N]
